# baseline (speedup 1.0000x reference)
import functools

import jax
import jax.numpy as jnp
from jax import lax
from jax.experimental import pallas as pl
from jax.experimental.pallas import tpu as pltpu
from jax.experimental.pallas import tpu_sc as plsc

D_MODEL_K = 2048
N_EXPERTS = 16
K_TOP = 2
N_TOK = 16384
BLOCK = 2048

NC, NS = 2, 16
NW = NC * NS
TOK_PER_TILE = N_TOK // NW
CHUNK_T = 128
N_CHUNKS = TOK_PER_TILE // CHUNK_T
CGROUPS = CHUNK_T // 16

_mesh = plsc.VectorSubcoreMesh(
    core_axis_name="c", subcore_axis_name="s",
    num_cores=NC, num_subcores=NS)


def _mm_body(x_ref, w_ref, b_ref, lg_ref):
    lg_ref[...] = jax.lax.dot_general(
        x_ref[...], w_ref[...],
        dimension_numbers=(((1,), (1,)), ((), ())),
        preferred_element_type=jnp.float32,
    ) + b_ref[...]


def _tc_logits(x, W, b2):
    n_tok = x.shape[0]
    return pl.pallas_call(
        _mm_body,
        grid=(n_tok // BLOCK,),
        in_specs=[
            pl.BlockSpec((BLOCK, D_MODEL_K), lambda i: (i, 0)),
            pl.BlockSpec((N_EXPERTS, D_MODEL_K), lambda i: (0, 0)),
            pl.BlockSpec((1, N_EXPERTS), lambda i: (0, 0)),
        ],
        out_specs=pl.BlockSpec((BLOCK, N_EXPERTS), lambda i: (i, 0)),
        out_shape=jax.ShapeDtypeStruct((n_tok, N_EXPERTS), jnp.float32),
        compiler_params=pltpu.CompilerParams(
            dimension_semantics=("arbitrary",),
        ),
    )(x, W, b2)


@functools.partial(
    pl.kernel,
    out_type=(
        jax.ShapeDtypeStruct((N_TOK, K_TOP), jnp.float32),
        jax.ShapeDtypeStruct((N_TOK, K_TOP), jnp.int32),
        jax.ShapeDtypeStruct((N_TOK, N_EXPERTS), jnp.float32),
    ),
    mesh=_mesh,
    scratch_types=[
        pltpu.VMEM((CHUNK_T, N_EXPERTS), jnp.float32),
        pltpu.VMEM((CHUNK_T, N_EXPERTS), jnp.float32),
        pltpu.VMEM((CHUNK_T, K_TOP), jnp.float32),
        pltpu.VMEM((CHUNK_T, K_TOP), jnp.float32),
        pltpu.VMEM((CHUNK_T, K_TOP), jnp.int32),
        pltpu.VMEM((CHUNK_T, K_TOP), jnp.int32),
        pltpu.SemaphoreType.DMA,
        pltpu.SemaphoreType.DMA,
        pltpu.SemaphoreType.DMA,
        pltpu.SemaphoreType.DMA,
        pltpu.SemaphoreType.DMA,
        pltpu.SemaphoreType.DMA,
        pltpu.SemaphoreType.DMA,
        pltpu.SemaphoreType.DMA,
    ],
    compiler_params=pltpu.CompilerParams(
        needs_layout_passes=False, use_tc_tiling_on_sc=True),
)
def _sc_route(lg_hbm, tw_hbm, ti_hbm, wt_hbm, lb0, lb1, tw0, tw1,
              ti0, ti1, si0, si1, sw0, sw1, st0, st1, so0, so1):
    wid = lax.axis_index("s") * NC + lax.axis_index("c")
    lbs, tws, tis = (lb0, lb1), (tw0, tw1), (ti0, ti1)
    in_sems, wt_sems = (si0, si1), (sw0, sw1)
    tw_sems, ti_sems = (st0, st1), (so0, so1)

    lane = lax.iota(jnp.int32, 16)
    neg_inf = jnp.full((16,), -jnp.inf, jnp.float32)
    zero_f = jnp.zeros((16,), jnp.float32)
    zero_i = jnp.zeros((16,), jnp.int32)
    one_i = jnp.full((16,), 1, jnp.int32)

    def make_group_body(lbuf, twb, tib):
      def group_body(g, carry):
        row = g * 16 + lane
        L = [plsc.load_gather(lbuf, [row, jnp.full((16,), e, jnp.int32)])
             for e in range(N_EXPERTS)]

        m = L[0]
        for e in range(1, N_EXPERTS):
            m = jnp.maximum(m, L[e])
        exps = [jnp.exp(L[e] - m) for e in range(N_EXPERTS)]
        s = exps[0]
        for e in range(1, N_EXPERTS):
            s = s + exps[e]
        r = 1.0 / s

        m1, i1 = L[0], zero_f
        m2, i2 = neg_inf, zero_f
        for e in range(1, N_EXPERTS):
            e_f = jnp.full((16,), float(e), jnp.float32)
            gt1 = L[e] > m1
            gt2 = L[e] > m2
            m2 = jnp.where(gt1, m1, jnp.where(gt2, L[e], m2))
            i2 = jnp.where(gt1, i1, jnp.where(gt2, e_f, i2))
            m1 = jnp.where(gt1, L[e], m1)
            i1 = jnp.where(gt1, e_f, i1)

        w1 = jnp.exp(m1 - m) * r
        w2 = jnp.exp(m2 - m) * r

        for e in range(N_EXPERTS):
            plsc.store_scatter(
                lbuf, [row, jnp.full((16,), e, jnp.int32)], exps[e] * r)
        plsc.store_scatter(twb, [row, zero_i], w1)
        plsc.store_scatter(twb, [row, one_i], w2)
        plsc.store_scatter(tib, [row, zero_i], i1.astype(jnp.int32))
        plsc.store_scatter(tib, [row, one_i], i2.astype(jnp.int32))
        return carry
      return group_body

    def chunk_base(c):
        return wid * TOK_PER_TILE + c * CHUNK_T

    in_copies = [None] * N_CHUNKS
    out_copies = [None] * N_CHUNKS
    for c in range(2):
        in_copies[c] = pltpu.async_copy(
            lg_hbm.at[pl.ds(chunk_base(c), CHUNK_T), :], lbs[c % 2],
            in_sems[c % 2])
    for c in range(N_CHUNKS):
        s = c % 2
        base = chunk_base(c)
        in_copies[c].wait()
        if c >= 2:
            _, ctw, cti = out_copies[c - 2]
            ctw.wait()
            cti.wait()
        lax.fori_loop(0, CGROUPS, make_group_body(lbs[s], tws[s], tis[s]),
                      jnp.int32(0))
        out_copies[c] = (
            pltpu.async_copy(lbs[s], wt_hbm.at[pl.ds(base, CHUNK_T), :],
                             wt_sems[s]),
            pltpu.async_copy(tws[s], tw_hbm.at[pl.ds(base, CHUNK_T), :],
                             tw_sems[s]),
            pltpu.async_copy(tis[s], ti_hbm.at[pl.ds(base, CHUNK_T), :],
                             ti_sems[s]),
        )
        if c + 2 < N_CHUNKS:
            out_copies[c][0].wait()
            in_copies[c + 2] = pltpu.async_copy(
                lg_hbm.at[pl.ds(chunk_base(c + 2), CHUNK_T), :], lbs[s],
                in_sems[s])
    for c in (N_CHUNKS - 2, N_CHUNKS - 1):
        cwt, ctw, cti = out_copies[c]
        cwt.wait()
        ctw.wait()
        cti.wait()


@functools.partial(jax.jit, static_argnames=())
def kernel(x, W, b):
    b2 = b.reshape(1, N_EXPERTS)
    logits = _tc_logits(x, W, b2)
    tw, ti, wts = _sc_route(logits)
    return (tw, ti, wts)
